# baseline (speedup 1.0000x reference)
import functools

import jax
import jax.numpy as jnp
from jax import lax
from jax.experimental import pallas as pl
from jax.experimental.pallas import tpu as pltpu
from jax.experimental.pallas import tpu_sc as plsc

_B = 1024
_K = 100000
_O = 64
_NC = 2
_NS = 16
_NW = _NC * _NS
_R = 8
_NG = _K // _R


def _sc_body(xT, wT, bias_h, out, xbuf, wbuf, acc, zbuf, idxv, bvm, shared):
    cid = lax.axis_index("c")
    sid = lax.axis_index("s")
    wid = sid * _NC + cid
    nch = jnp.where(wid < _NG % _NW, _NG // _NW + 1, _NG // _NW)

    pltpu.sync_copy(bias_h, bvm)

    bsel = jnp.where(wid == 0, jnp.float32(1.0), jnp.float32(0.0))

    def _init_row(r, _):
        for j in range(_O // 16):
            acc[r, pl.ds(j * 16, 16)] = bvm[pl.ds(j * 16, 16)] * bsel
        return 0

    lax.fori_loop(0, _B, _init_row, 0)

    lanes = lax.iota(jnp.int32, 16)

    def _chunk(c, _):
        base = (wid + c * _NW) * _R
        pltpu.sync_copy(xT.at[pl.ds(base, _R)], xbuf)
        pltpu.sync_copy(wT.at[pl.ds(base, _R)], wbuf)

        def _row(r, _):
            def _vreg(g, _):
                xv = xbuf[r, pl.ds(g * 16, 16)]
                m0 = xv != 0.0

                @pl.when(jnp.any(m0))
                def _extract():
                    def _cond(m):
                        return jnp.any(m)

                    def _step(m):
                        ffs = plsc.all_reduce_ffs(m)
                        val = lax.gather(
                            xv, ffs[:, None],
                            dimension_numbers=lax.GatherDimensionNumbers(
                                offset_dims=(), collapsed_slice_dims=(0,),
                                start_index_map=(0,)),
                            slice_sizes=(1,),
                            mode=lax.GatherScatterMode.PROMISE_IN_BOUNDS)
                        row = g * 16 + ffs
                        for j in range(_O // 16):
                            sl = pl.ds(j * 16, 16)
                            plsc.addupdate_scatter(
                                acc, [row, j * 16 + lanes],
                                val * wbuf[r, sl])
                        return m & (lanes != ffs)

                    lax.while_loop(_cond, _step, m0)

                return 0

            lax.fori_loop(0, _B // 16, _vreg, 0)
            return 0

        lax.fori_loop(0, _R, _row, 0)
        return 0

    lax.fori_loop(0, nch, _chunk, 0)

    def _zrow(r, _):
        for j in range(_O // 16):
            zbuf[r, pl.ds(j * 16, 16)] = jnp.zeros((16,), jnp.float32)
        return 0

    lax.fori_loop(0, _O, _zrow, 0)

    def _irow(r, _):
        idxv[pl.ds(r * 16, 16)] = lanes + r * 16
        return 0

    lax.fori_loop(0, _B // 16, _irow, 0)

    rows = _B // _NS
    pltpu.sync_copy(zbuf, shared.at[pl.ds(sid * rows, rows)])
    plsc.subcore_barrier()
    pltpu.sync_copy(acc, shared.at[idxv], add=True)
    plsc.subcore_barrier()
    pltpu.sync_copy(shared.at[pl.ds(sid * rows, rows)], zbuf)
    pltpu.sync_copy(zbuf, out.at[cid, pl.ds(sid * rows, rows)])


_sc_call = functools.partial(
    pl.kernel,
    mesh=plsc.VectorSubcoreMesh(core_axis_name="c", subcore_axis_name="s"),
    compiler_params=pltpu.CompilerParams(
        needs_layout_passes=False, use_tc_tiling_on_sc=False),
    out_type=jax.ShapeDtypeStruct((_NC, _B, _O), jnp.float32),
    scratch_types=[
        pltpu.VMEM((_R, _B), jnp.float32),
        pltpu.VMEM((_R, _O), jnp.float32),
        pltpu.VMEM((_B, _O), jnp.float32),
        pltpu.VMEM((_B // _NS, _O), jnp.float32),
        pltpu.VMEM((_B,), jnp.int32),
        pltpu.VMEM((_O,), jnp.float32),
        pltpu.VMEM_SHARED((_B, _O), jnp.float32),
    ],
)


def kernel(input, weight, bias):
    parts = _sc_call(_sc_body)(input.T, weight.T, bias)
    return parts[0] + parts[1]


# speedup vs baseline: 14.4909x; 14.4909x over previous
import functools

import jax
import jax.numpy as jnp
from jax import lax
from jax.experimental import pallas as pl
from jax.experimental.pallas import tpu as pltpu
from jax.experimental.pallas import tpu_sc as plsc

_B = 1024
_K = 100000
_O = 64
_KBLK = 2048
_NSTEP = 48
_KSC = _NSTEP * _KBLK
_NC = 2
_NS = 16
_NW = _NC * _NS
_R = 8
_NG = (_K - _KSC) // _R


def _mm_body(x_ref, w_ref, b_ref, o_ref):
    k = pl.program_id(0)

    @pl.when(k == 0)
    def _init():
        o_ref[...] = jnp.broadcast_to(b_ref[...], (_O, _B))

    o_ref[...] += jax.lax.dot_general(
        w_ref[...], x_ref[...], (((0,), (0,)), ((), ())),
        preferred_element_type=jnp.float32)


def _tc_call(xT, wT, bias):
    return pl.pallas_call(
        _mm_body,
        grid=(_NSTEP,),
        in_specs=[
            pl.BlockSpec((_KBLK, _B), lambda k: (k, 0)),
            pl.BlockSpec((_KBLK, _O), lambda k: (k, 0)),
            pl.BlockSpec((_O, 1), lambda k: (0, 0)),
        ],
        out_specs=pl.BlockSpec((_O, _B), lambda k: (0, 0)),
        out_shape=jax.ShapeDtypeStruct((_O, _B), jnp.float32),
        compiler_params=pltpu.CompilerParams(
            dimension_semantics=("arbitrary",),
        ),
    )(xT, wT, bias.reshape(_O, 1))


def _sc_body(xT, wT, out, xbuf, wbuf, acc, zbuf, idxv, shared):
    cid = lax.axis_index("c")
    sid = lax.axis_index("s")
    wid = sid * _NC + cid
    nch = jnp.where(wid < _NG % _NW, _NG // _NW + 1, _NG // _NW)

    def _init_row(r, _):
        for j in range(_O // 16):
            acc[r, pl.ds(j * 16, 16)] = jnp.zeros((16,), jnp.float32)
        return 0

    lax.fori_loop(0, _B, _init_row, 0)

    lanes = lax.iota(jnp.int32, 16)

    def _chunk(c, _):
        base = _KSC + (wid + c * _NW) * _R
        pltpu.sync_copy(xT.at[pl.ds(base, _R)], xbuf)
        pltpu.sync_copy(wT.at[pl.ds(base, _R)], wbuf)

        def _row(r, _):
            def _vreg(g, _):
                xv = xbuf[r, pl.ds(g * 16, 16)]
                m0 = xv != 0.0

                @pl.when(jnp.any(m0))
                def _extract():
                    def _cond(m):
                        return jnp.any(m)

                    def _step(m):
                        ffs = plsc.all_reduce_ffs(m)
                        val = lax.gather(
                            xv, ffs[:, None],
                            dimension_numbers=lax.GatherDimensionNumbers(
                                offset_dims=(), collapsed_slice_dims=(0,),
                                start_index_map=(0,)),
                            slice_sizes=(1,),
                            mode=lax.GatherScatterMode.PROMISE_IN_BOUNDS)
                        row = g * 16 + ffs
                        for j in range(_O // 16):
                            sl = pl.ds(j * 16, 16)
                            plsc.addupdate_scatter(
                                acc, [row, j * 16 + lanes],
                                val * wbuf[r, sl])
                        return m & (lanes != ffs)

                    lax.while_loop(_cond, _step, m0)

                return 0

            lax.fori_loop(0, _B // 16, _vreg, 0)
            return 0

        lax.fori_loop(0, _R, _row, 0)
        return 0

    lax.fori_loop(0, nch, _chunk, 0)

    def _zrow(r, _):
        for j in range(_O // 16):
            zbuf[r, pl.ds(j * 16, 16)] = jnp.zeros((16,), jnp.float32)
        return 0

    lax.fori_loop(0, _B // _NS, _zrow, 0)

    def _irow(r, _):
        idxv[pl.ds(r * 16, 16)] = lanes + r * 16
        return 0

    lax.fori_loop(0, _B // 16, _irow, 0)

    rows = _B // _NS
    pltpu.sync_copy(zbuf, shared.at[pl.ds(sid * rows, rows)])
    plsc.subcore_barrier()
    pltpu.sync_copy(acc, shared.at[idxv], add=True)
    plsc.subcore_barrier()
    pltpu.sync_copy(shared.at[pl.ds(sid * rows, rows)], zbuf)
    pltpu.sync_copy(zbuf, out.at[cid, pl.ds(sid * rows, rows)])


_sc_call = functools.partial(
    pl.kernel,
    mesh=plsc.VectorSubcoreMesh(core_axis_name="c", subcore_axis_name="s"),
    compiler_params=pltpu.CompilerParams(
        needs_layout_passes=False, use_tc_tiling_on_sc=False),
    out_type=jax.ShapeDtypeStruct((_NC, _B, _O), jnp.float32),
    scratch_types=[
        pltpu.VMEM((_R, _B), jnp.float32),
        pltpu.VMEM((_R, _O), jnp.float32),
        pltpu.VMEM((_B, _O), jnp.float32),
        pltpu.VMEM((_B // _NS, _O), jnp.float32),
        pltpu.VMEM((_B,), jnp.int32),
        pltpu.VMEM_SHARED((_B, _O), jnp.float32),
    ],
)


def kernel(input, weight, bias):
    xT = input.T
    wT = weight.T
    parts = _sc_call(_sc_body)(xT, wT)
    out_t = _tc_call(xT, wT, bias)
    return out_t.T + parts[0] + parts[1]


# speedup vs baseline: 30.1481x; 2.0805x over previous
import functools

import jax
import jax.numpy as jnp
from jax import lax
from jax.experimental import pallas as pl
from jax.experimental.pallas import tpu as pltpu
from jax.experimental.pallas import tpu_sc as plsc

_B = 1024
_K = 100000
_O = 64
_KBLK = 2048
_NBLK = 48
_SCBLK = 2
_K0SC = (_NBLK - _SCBLK) * _KBLK
_KSC = _SCBLK * _KBLK
_TCSTEP = _NBLK - _SCBLK + 1
_TAIL = _K - _NBLK * _KBLK
_NC = 2
_NS = 16
_NW = _NC * _NS
_CHUNK = _KSC // _NW
_SR = 16


def _mm_body(x_ref, w_ref, b_ref, o_ref):
    k = pl.program_id(0)

    @pl.when(k == 0)
    def _init():
        o_ref[...] = jnp.broadcast_to(b_ref[...], (_O, _B))

    @pl.when(k < _TCSTEP - 1)
    def _full():
        o_ref[...] += jax.lax.dot_general(
            w_ref[...], x_ref[...], (((1,), (0,)), ((), ())),
            preferred_element_type=jnp.float32)

    @pl.when(k == _TCSTEP - 1)
    def _tail():
        x = jnp.where(
            jax.lax.broadcasted_iota(jnp.int32, (_KBLK, _B), 0) < _TAIL,
            x_ref[...], 0.0)
        w = jnp.where(
            jax.lax.broadcasted_iota(jnp.int32, (_O, _KBLK), 1) < _TAIL,
            w_ref[...], 0.0)
        o_ref[...] += jax.lax.dot_general(
            w, x, (((1,), (0,)), ((), ())),
            preferred_element_type=jnp.float32)


def _blk(k):
    return jnp.where(k < _TCSTEP - 1, k, _NBLK)


def _tc_call(xT, weight, bias):
    return pl.pallas_call(
        _mm_body,
        grid=(_TCSTEP,),
        in_specs=[
            pl.BlockSpec((_KBLK, _B), lambda k: (_blk(k), 0)),
            pl.BlockSpec((_O, _KBLK), lambda k: (0, _blk(k))),
            pl.BlockSpec((_O, 1), lambda k: (0, 0)),
        ],
        out_specs=pl.BlockSpec((_O, _B), lambda k: (0, 0)),
        out_shape=jax.ShapeDtypeStruct((_O, _B), jnp.float32),
        compiler_params=pltpu.CompilerParams(
            dimension_semantics=("arbitrary",),
        ),
    )(xT, weight, bias.reshape(_O, 1))


_BG = 128
_KQ = _KSC // _NC // 2
_NCH = _KQ // _CHUNK


def _sc_body(xT, weight, out, xbuf, wbuf, acc, tbuf, cval, cidx, shared):
    cid = lax.axis_index("c")
    sid = lax.axis_index("s")
    bg = sid % 8
    kq = sid // 8
    b0 = bg * _BG
    kbase = _K0SC + cid * (2 * _KQ) + kq * _KQ

    lanes = lax.iota(jnp.int32, 16)

    def _acc_zero(r, _):
        for g in range(_BG // 16):
            acc[r, pl.ds(g * 16, 16)] = jnp.zeros((16,), jnp.float32)
        return 0

    lax.fori_loop(0, _O, _acc_zero, 0)

    def _chunkloop(ch, _):
        cb = kbase + ch * _CHUNK
        pltpu.sync_copy(weight.at[:, pl.ds(cb, _CHUNK)], wbuf)
        pltpu.sync_copy(xT.at[pl.ds(cb, _CHUNK), pl.ds(b0, _BG)], xbuf)

        def _row(r, _):
            def _scan(g, off):
                xv = xbuf[r, pl.ds(g * 16, 16)]
                m = xv != 0.0
                plsc.store_compressed(cval.at[pl.ds(off, 16)], xv, mask=m)
                plsc.store_compressed(cidx.at[pl.ds(off, 16)],
                                      g * 16 + lanes, mask=m)
                return off + plsc.all_reduce_population_count(m)[0]

            n = lax.fori_loop(0, _BG // 16, _scan, jnp.int32(0))

            @pl.when(n > 0)
            def _flush():
                ksp = jnp.full((16,), r, jnp.int32)
                wv = [plsc.load_gather(wbuf, [j * 16 + lanes, ksp])
                      for j in range(_O // 16)]

                def _ent(i, _):
                    isp = jnp.full((16,), i, jnp.int32)
                    bsp = plsc.load_gather(cidx, [isp])
                    vsp = plsc.load_gather(cval, [isp])
                    for j in range(_O // 16):
                        plsc.addupdate_scatter(
                            acc, [j * 16 + lanes, bsp], vsp * wv[j])
                    return 0

                lax.fori_loop(0, n, _ent, 0)

            return 0

        lax.fori_loop(0, _CHUNK, _row, 0)
        return 0

    lax.fori_loop(0, _NCH, _chunkloop, 0)

    @pl.when(kq == 1)
    def _push():
        pltpu.sync_copy(acc, shared.at[bg])

    plsc.subcore_barrier()

    @pl.when(kq == 0)
    def _merge():
        pltpu.sync_copy(shared.at[bg], tbuf)

        def _addrow(r, _):
            for g in range(_BG // 16):
                sl = pl.ds(g * 16, 16)
                acc[r, sl] = acc[r, sl] + tbuf[r, sl]
            return 0

        lax.fori_loop(0, _O, _addrow, 0)
        pltpu.sync_copy(acc, out.at[cid, :, pl.ds(b0, _BG)])


_sc_call = functools.partial(
    pl.kernel,
    mesh=plsc.VectorSubcoreMesh(core_axis_name="c", subcore_axis_name="s"),
    compiler_params=pltpu.CompilerParams(needs_layout_passes=False),
    out_type=jax.ShapeDtypeStruct((_NC, _O, _B), jnp.float32),
    scratch_types=[
        pltpu.VMEM((_CHUNK, _BG), jnp.float32),
        pltpu.VMEM((_O, _CHUNK), jnp.float32),
        pltpu.VMEM((_O, _BG), jnp.float32),
        pltpu.VMEM((_O, _BG), jnp.float32),
        pltpu.VMEM((_BG + 16,), jnp.float32),
        pltpu.VMEM((_BG + 16,), jnp.int32),
        pltpu.VMEM_SHARED((8, _O, _BG), jnp.float32),
    ],
)


def kernel(input, weight, bias):
    xT = input.T
    parts = _sc_call(_sc_body)(xT, weight)
    out_t = _tc_call(xT, weight, bias)
    return (out_t + parts[0] + parts[1]).T


# speedup vs baseline: 56.1340x; 1.8619x over previous
import jax
import jax.numpy as jnp
from jax.experimental import pallas as pl
from jax.experimental.pallas import tpu as pltpu

_B = 1024
_K = 100000
_O = 64
_KBLK = 4096
_NSTEP = (_K + _KBLK - 1) // _KBLK
_DIMNUMS = (((1,), (0,)), ((), ()))


def _mm_body(x_ref, w_ref, b_ref, o_ref):
    k = pl.program_id(0)

    @pl.when(k == 0)
    def _init():
        o_ref[...] = jnp.broadcast_to(b_ref[...], (_O, _B))

    @pl.when(k < _NSTEP - 1)
    def _full():
        o_ref[...] += jax.lax.dot_general(
            w_ref[...], x_ref[...], _DIMNUMS,
            preferred_element_type=jnp.float32)

    @pl.when(k == _NSTEP - 1)
    def _tail():
        valid = _K - (_NSTEP - 1) * _KBLK
        x = jnp.where(
            jax.lax.broadcasted_iota(jnp.int32, (_KBLK, _B), 0) < valid,
            x_ref[...], 0.0)
        w = jnp.where(
            jax.lax.broadcasted_iota(jnp.int32, (_O, _KBLK), 1) < valid,
            w_ref[...], 0.0)
        o_ref[...] += jax.lax.dot_general(
            w, x, _DIMNUMS, preferred_element_type=jnp.float32)


def kernel(input, weight, bias):
    out_t = pl.pallas_call(
        _mm_body,
        grid=(_NSTEP,),
        in_specs=[
            pl.BlockSpec((_KBLK, _B), lambda k: (k, 0)),
            pl.BlockSpec((_O, _KBLK), lambda k: (0, k)),
            pl.BlockSpec((_O, 1), lambda k: (0, 0)),
        ],
        out_specs=pl.BlockSpec((_O, _B), lambda k: (0, 0)),
        out_shape=jax.ShapeDtypeStruct((_O, _B), jnp.float32),
        compiler_params=pltpu.CompilerParams(
            dimension_semantics=("arbitrary",),
        ),
    )(input.T, weight, bias.reshape(_O, 1))
    return out_t.T
